# baseline (speedup 1.0000x reference)
import jax
import jax.numpy as jnp
from jax import lax
from jax.experimental import pallas as pl
from jax.experimental.pallas import tpu as pltpu
from jax.experimental.pallas import tpu_sc as plsc

N_ROWS = 16384
N_COLS = 1000
BLOCK_ROWS = 128
NUM_BLOCKS = N_ROWS // BLOCK_ROWS
NUM_BINS = 5
LANES = 16
INV_TOTAL = 1.0 / float(N_ROWS * N_COLS)
BIG_IDX = 2 ** 30
PART_ROWS = NUM_BINS * 3


def _tc_rowstats_body(x_ref, lab_ref, conf_ref, acc_ref, bits_ref, cnt_ref):
    x = x_ref[...]
    rowmax = jnp.max(x, axis=1)
    conf_ref[...] = rowmax
    col = lax.broadcasted_iota(jnp.int32, x.shape, 1)
    ismax = x == rowmax[:, None]
    pred = jnp.min(jnp.where(ismax, col, BIG_IDX), axis=1)
    acc_ref[...] = (pred == lab_ref[...]).astype(jnp.float32)
    s = [jnp.sum((x > jnp.float32(k)).astype(jnp.float32), axis=1)
         for k in range(NUM_BINS + 1)]
    bits = jnp.zeros((BLOCK_ROWS,), jnp.int32)
    lane = lax.broadcasted_iota(jnp.int32, (1, 1, LANES), 2)
    cnt_row = jnp.zeros((1, 1, LANES), jnp.float32)
    for k in range(NUM_BINS):
        ck = s[k] - s[k + 1]
        bits = bits | ((ck > 0.0).astype(jnp.int32) << k)
        cnt_row = cnt_row + jnp.where(lane == k, jnp.sum(ck), 0.0)
    bits_ref[...] = bits
    cnt_ref[...] = cnt_row


_stage1 = pl.pallas_call(
    _tc_rowstats_body,
    grid=(NUM_BLOCKS,),
    in_specs=[
        pl.BlockSpec((BLOCK_ROWS, N_COLS), lambda i: (i, 0)),
        pl.BlockSpec((BLOCK_ROWS,), lambda i: (i,)),
    ],
    out_specs=[
        pl.BlockSpec((BLOCK_ROWS,), lambda i: (i,)),
        pl.BlockSpec((BLOCK_ROWS,), lambda i: (i,)),
        pl.BlockSpec((BLOCK_ROWS,), lambda i: (i,)),
        pl.BlockSpec((1, 1, LANES), lambda i: (i, 0, 0)),
    ],
    out_shape=[
        jax.ShapeDtypeStruct((N_ROWS,), jnp.float32),
        jax.ShapeDtypeStruct((N_ROWS,), jnp.float32),
        jax.ShapeDtypeStruct((N_ROWS,), jnp.int32),
        jax.ShapeDtypeStruct((NUM_BLOCKS, 1, LANES), jnp.float32),
    ],
)


def _make_sc_body(num_cores, rows_per_worker):
    def _sc_binstats_body(conf_hbm, acc_hbm, bits_hbm, out_hbm,
                          conf_v, acc_v, bits_v, part_v):
        c = lax.axis_index("c")
        s = lax.axis_index("s")
        wid = s * num_cores + c
        base = wid * rows_per_worker
        pltpu.sync_copy(conf_hbm.at[pl.ds(base, rows_per_worker)], conf_v)
        pltpu.sync_copy(acc_hbm.at[pl.ds(base, rows_per_worker)], acc_v)
        pltpu.sync_copy(bits_hbm.at[pl.ds(base, rows_per_worker)], bits_v)

        zero = jnp.zeros((LANES,), jnp.float32)

        def row_step(j, carry):
            off = pl.multiple_of(j * LANES, LANES)
            cf = conf_v[pl.ds(off, LANES)]
            ac = acc_v[pl.ds(off, LANES)]
            bt = bits_v[pl.ds(off, LANES)]
            out = []
            for k in range(NUM_BINS):
                cntk, ak, sk = carry[k]
                mf = ((bt >> k) & 1).astype(jnp.float32)
                out.append((cntk + mf, ak + mf * ac, sk + mf * cf))
            return tuple(out)

        init = tuple((zero, zero, zero) for _ in range(NUM_BINS))
        stats = lax.fori_loop(0, rows_per_worker // LANES, row_step, init)
        for k in range(NUM_BINS):
            cntk, ak, sk = stats[k]
            part_v[3 * k + 0, :] = cntk
            part_v[3 * k + 1, :] = ak
            part_v[3 * k + 2, :] = sk
        pltpu.sync_copy(part_v, out_hbm.at[wid])

    return _sc_binstats_body


_stage2_cache = []


def _get_stage2():
    if not _stage2_cache:
        info = plsc.get_sparse_core_info()
        num_workers = info.num_cores * info.num_subcores
        rows_per_worker = N_ROWS // num_workers
        _stage2_cache.append((pl.kernel(
            _make_sc_body(info.num_cores, rows_per_worker),
            mesh=plsc.VectorSubcoreMesh(core_axis_name="c",
                                        subcore_axis_name="s"),
            out_type=jax.ShapeDtypeStruct((num_workers, PART_ROWS, LANES),
                                          jnp.float32),
            scratch_types=[
                pltpu.VMEM((rows_per_worker,), jnp.float32),
                pltpu.VMEM((rows_per_worker,), jnp.float32),
                pltpu.VMEM((rows_per_worker,), jnp.int32),
                pltpu.VMEM((PART_ROWS, LANES), jnp.float32),
            ],
        ), num_workers))
    return _stage2_cache[0]


def _tc_combine_body(part_ref, cnt_ref, out_ref):
    p = part_ref[...]
    tot = jnp.sum(p, axis=(0, 2))
    ecnt = jnp.sum(cnt_ref[...], axis=(0, 1))
    ece = jnp.float32(0.0)
    for k in range(NUM_BINS):
        cnt = tot[3 * k + 0]
        asum = tot[3 * k + 1]
        csum = tot[3 * k + 2]
        ec = ecnt[k]
        safe = jnp.maximum(cnt, 1.0)
        term = jnp.abs(csum / safe - asum / safe) * (ec * jnp.float32(INV_TOTAL))
        ece = ece + jnp.where(ec > 0.0, term, 0.0)
    out_ref[...] = jnp.full((8, 128), ece, jnp.float32)


def _make_stage3(num_workers):
    return pl.pallas_call(
        _tc_combine_body,
        in_specs=[
            pl.BlockSpec((num_workers, PART_ROWS, LANES),
                         lambda: (0, 0, 0)),
            pl.BlockSpec((NUM_BLOCKS, 1, LANES), lambda: (0, 0, 0)),
        ],
        out_specs=pl.BlockSpec((8, 128), lambda: (0, 0)),
        out_shape=jax.ShapeDtypeStruct((8, 128), jnp.float32),
    )


def kernel(logits, labels):
    labels = labels.astype(jnp.int32)
    conf, acc, bits, cnts = _stage1(logits, labels)
    stage2, num_workers = _get_stage2()
    parts = stage2(conf, acc, bits)
    out = _make_stage3(num_workers)(parts, cnts)
    return out[0, :1]
